# baseline (speedup 1.0000x reference)
import functools

import jax
import jax.numpy as jnp
from jax import lax
from jax.experimental import pallas as pl
from jax.experimental.pallas import tpu as pltpu

_F32 = jnp.float32


def _fps_kernel(xyz_ref, ox_ref, oy_ref, oz_ref):
    _, B, N = xyz_ref.shape
    S = ox_ref.shape[1]
    col = lax.broadcasted_iota(jnp.int32, (B, N), 1)
    colS = lax.broadcasted_iota(jnp.int32, (B, S), 1)
    X = xyz_ref[0]
    Y = xyz_ref[1]
    Z = xyz_ref[2]

    def step(t, carry):
        dist, far = carry
        onehot = col == far
        cx = jnp.sum(jnp.where(onehot, X, 0.0), axis=1, keepdims=True)
        cy = jnp.sum(jnp.where(onehot, Y, 0.0), axis=1, keepdims=True)
        cz = jnp.sum(jnp.where(onehot, Z, 0.0), axis=1, keepdims=True)
        emit = colS == t
        ox_ref[...] = jnp.where(emit, cx, ox_ref[...])
        oy_ref[...] = jnp.where(emit, cy, oy_ref[...])
        oz_ref[...] = jnp.where(emit, cz, oz_ref[...])
        dx = X - cx
        dy = Y - cy
        dz = Z - cz
        dist = jnp.minimum(dist, (dx * dx + dy * dy) + dz * dz)
        dmax = jnp.max(dist, axis=1, keepdims=True)
        far = jnp.min(jnp.where(dist == dmax, col, N), axis=1, keepdims=True)
        return dist, far

    lax.fori_loop(0, S, step,
                  (jnp.full((B, N), 1e10, _F32), jnp.zeros((B, 1), jnp.int32)))


def _fps_new_xyz(xyzT, npoint, interpret=False):
    B, _, N = xyzT.shape
    shp = jax.ShapeDtypeStruct((B, npoint), _F32)
    ox, oy, oz = pl.pallas_call(
        _fps_kernel,
        out_shape=(shp, shp, shp),
        interpret=interpret,
    )(jnp.transpose(xyzT, (1, 0, 2)))
    return jnp.stack([ox, oy, oz], axis=1)


def _sum11(x):
    return jnp.sum(jnp.sum(x, axis=1, keepdims=True), axis=0, keepdims=True)


def _gn_apply(x, mu, var, g, beta, eps=1e-5):
    return (x - mu) / jnp.sqrt(var + eps) * g + beta


def _sigmoid(x):
    return 1.0 / (1.0 + jnp.exp(-x))


def _sa_body_kernel(N, S, K, C, O, Cout, bw,
                    xyzN_ref, xyzT_ref, ptsT_ref, nxT_ref, nxS_ref,
                    wm_ref, bm_ref, gm_ref, bem_ref,
                    ww1_ref, bw1_ref, gw1_ref, bew1_ref,
                    ww2_ref, bw2_ref, gw2_ref, bew2_ref,
                    ww3_ref, bw3_ref, gw3_ref, bew3_ref,
                    wd1_ref, bd1_ref, gd1_ref, bed1_ref,
                    wd2_ref, bd2_ref, gd2_ref, bed2_ref,
                    wd3_ref, bd3_ref, gd3_ref, bed3_ref,
                    lw_ref, lb_ref, gl_ref, bel_ref,
                    out_ref,
                    sqr_ref, buf_ref, hbuf_ref, w1b_ref, w2b_ref, w3b_ref,
                    d1b_ref, d2b_ref, d3b_ref, acc_ref):
    Ct = C + 4
    xyzN = xyzN_ref[0]
    X = xyzT_ref[0]
    P = ptsT_ref[0]
    NX = nxT_ref[0]

    a_row = jnp.sum(X * X, axis=0, keepdims=True)
    a_col = jnp.sum(xyzN * xyzN, axis=1, keepdims=True)
    Gm = jnp.dot(xyzN, X, preferred_element_type=_F32)
    sqr_full = (a_col + a_row) - 2.0 * Gm
    gk = jnp.exp(-sqr_full / (2.0 * bw * bw)) / (2.5 * bw)
    invd = N / jnp.sum(gk, axis=0, keepdims=True)

    nxS = nxS_ref[0]
    s_col = jnp.sum(nxS * nxS, axis=1, keepdims=True)
    Gn = lax.dot_general(nxS, xyzN, (((1,), (1,)), ((), ())),
                         preferred_element_type=_F32)
    sqr_ref[...] = jnp.transpose((s_col + a_row) - 2.0 * Gn)

    TT = jnp.concatenate([X, P, invd], axis=0)
    rowN = lax.broadcasted_iota(jnp.int32, (N, S), 0)

    def gather_step(j, gdmax):
        sq = sqr_ref[...]
        m = jnp.min(sq, axis=0, keepdims=True)
        first = jnp.min(jnp.where(sq == m, rowN, N), axis=0, keepdims=True)
        selb = rowN == first
        Gj = jnp.dot(TT, selb.astype(_F32), preferred_element_type=_F32)
        buf_ref[pl.ds(j, 1)] = Gj[None]
        sqr_ref[...] = jnp.where(selb, 1e30, sq)
        return jnp.maximum(gdmax, Gj[Ct - 1:Ct, :])

    gdmax = lax.fori_loop(0, K, gather_step, jnp.full((1, S), -1e30, _F32), unroll=4)

    wm = wm_ref[...]
    bm = bm_ref[...]
    ww1 = ww1_ref[...]
    bw1 = bw1_ref[...]
    wd1 = wd1_ref[...]
    bd1 = bd1_ref[...]

    z = jnp.zeros((1, 1), _F32)

    def loopA(j, cr):
        sh, sh2, sw, sw2, sd, sd2 = cr
        Gj = buf_ref[pl.ds(j, 1)][0]
        xn = Gj[0:3, :] - NX
        npj = jnp.concatenate([xn, Gj[3:3 + C, :]], axis=0)
        hj = jnp.dot(wm, npj, preferred_element_type=_F32) + bm
        hbuf_ref[pl.ds(j, 1)] = hj[None]
        w1j = jnp.dot(ww1, xn, preferred_element_type=_F32) + bw1
        w1b_ref[pl.ds(j, 1)] = w1j[None]
        dscj = Gj[Ct - 1:Ct, :] / gdmax
        d1j = wd1 * dscj + bd1
        d1b_ref[pl.ds(j, 1)] = d1j[None]
        return (sh + _sum11(hj), sh2 + _sum11(hj * hj),
                sw + _sum11(w1j), sw2 + _sum11(w1j * w1j),
                sd + _sum11(d1j), sd2 + _sum11(d1j * d1j))

    sh, sh2, sw, sw2, sd, sd2 = lax.fori_loop(0, K, loopA, (z, z, z, z, z, z), unroll=8)
    ch = float(O * K * S)
    mu_h = sh / ch
    var_h = sh2 / ch - mu_h * mu_h
    cw1 = float(8 * K * S)
    mu_w1 = sw / cw1
    var_w1 = sw2 / cw1 - mu_w1 * mu_w1
    cd1 = float(16 * K * S)
    mu_d1 = sd / cd1
    var_d1 = sd2 / cd1 - mu_d1 * mu_d1

    ww2 = ww2_ref[...]
    bw2 = bw2_ref[...]
    wd2 = wd2_ref[...]
    bd2 = bd2_ref[...]
    gw1 = gw1_ref[...]
    bew1 = bew1_ref[...]
    gd1 = gd1_ref[...]
    bed1 = bed1_ref[...]

    def loopB(j, cr):
        sw_, sw2_, sd_, sd2_ = cr
        w1j = jnp.maximum(_gn_apply(w1b_ref[pl.ds(j, 1)][0], mu_w1, var_w1, gw1, bew1), 0.0)
        w2j = jnp.dot(ww2, w1j, preferred_element_type=_F32) + bw2
        w2b_ref[pl.ds(j, 1)] = w2j[None]
        d1j = jnp.maximum(_gn_apply(d1b_ref[pl.ds(j, 1)][0], mu_d1, var_d1, gd1, bed1), 0.0)
        d2j = jnp.dot(wd2, d1j, preferred_element_type=_F32) + bd2
        d2b_ref[pl.ds(j, 1)] = d2j[None]
        return (sw_ + _sum11(w2j), sw2_ + _sum11(w2j * w2j),
                sd_ + _sum11(d2j), sd2_ + _sum11(d2j * d2j))

    sw, sw2, sd, sd2 = lax.fori_loop(0, K, loopB, (z, z, z, z), unroll=8)
    cw2 = float(8 * K * S)
    mu_w2 = sw / cw2
    var_w2 = sw2 / cw2 - mu_w2 * mu_w2
    cd2 = float(8 * K * S)
    mu_d2 = sd / cd2
    var_d2 = sd2 / cd2 - mu_d2 * mu_d2

    ww3 = ww3_ref[...]
    bw3 = bw3_ref[...]
    wd3 = wd3_ref[...]
    bd3 = bd3_ref[...]
    gw2 = gw2_ref[...]
    bew2 = bew2_ref[...]
    gd2 = gd2_ref[...]
    bed2 = bed2_ref[...]

    def loopC(j, cr):
        sw_, sw2_, sd_, sd2_ = cr
        w2j = jnp.maximum(_gn_apply(w2b_ref[pl.ds(j, 1)][0], mu_w2, var_w2, gw2, bew2), 0.0)
        w3j = jnp.dot(ww3, w2j, preferred_element_type=_F32) + bw3
        w3b_ref[pl.ds(j, 1)] = w3j[None]
        d2j = jnp.maximum(_gn_apply(d2b_ref[pl.ds(j, 1)][0], mu_d2, var_d2, gd2, bed2), 0.0)
        d3j = jnp.dot(wd3, d2j, preferred_element_type=_F32) + bd3
        d3b_ref[pl.ds(j, 1)] = d3j[None]
        return (sw_ + _sum11(w3j), sw2_ + _sum11(w3j * w3j),
                sd_ + _sum11(d3j), sd2_ + _sum11(d3j * d3j))

    sw, sw2, sd, sd2 = lax.fori_loop(0, K, loopC, (z, z, z, z), unroll=8)
    cw3 = float(16 * K * S)
    mu_w3 = sw / cw3
    var_w3 = sw2 / cw3 - mu_w3 * mu_w3
    cd3 = float(1 * K * S)
    mu_d3 = sd / cd3
    var_d3 = sd2 / cd3 - mu_d3 * mu_d3

    gm = gm_ref[...]
    bem = bem_ref[...]
    gw3 = gw3_ref[...]
    bew3 = bew3_ref[...]
    gd3 = gd3_ref[...]
    bed3 = bed3_ref[...]

    acc_ref[...] = jnp.zeros((O, 16, S), _F32)

    def loopD(j, _):
        hj = jnp.maximum(_gn_apply(hbuf_ref[pl.ds(j, 1)][0], mu_h, var_h, gm, bem), 0.0)
        d3j = _sigmoid(_gn_apply(d3b_ref[pl.ds(j, 1)][0], mu_d3, var_d3, gd3, bed3))
        xj = hj * d3j
        wtj = jnp.maximum(_gn_apply(w3b_ref[pl.ds(j, 1)][0], mu_w3, var_w3, gw3, bew3), 0.0)
        acc_ref[...] += xj[:, None, :] * wtj[None, :, :]
        return 0

    lax.fori_loop(0, K, loopD, 0, unroll=8)

    flat = acc_ref[...].reshape(O * 16, S)
    lout = jnp.dot(lw_ref[...], flat, preferred_element_type=_F32) + lb_ref[...]
    mu = jnp.sum(lout, axis=(0, 1), keepdims=True)[0] / float(Cout * S)
    xc = lout - mu
    var = jnp.sum(xc * xc, axis=(0, 1), keepdims=True)[0] / float(Cout * S)
    out_ref[0] = jnp.maximum(xc / jnp.sqrt(var + 1e-5) * gl_ref[...] + bel_ref[...], 0.0)


def _col(v):
    return v[:, None]


def _sa_weight_args(p, O, Cout):
    m = p['mlp'][0]
    w1, w2, w3 = p['weight']
    d1, d2, d3 = p['density']
    return [
        m['w'], _col(m['b']), _col(m['g']), _col(m['beta']),
        w1['w'], _col(w1['b']), _col(w1['g']), _col(w1['beta']),
        w2['w'], _col(w2['b']), _col(w2['g']), _col(w2['beta']),
        w3['w'], _col(w3['b']), _col(w3['g']), _col(w3['beta']),
        d1['w'], _col(d1['b']), _col(d1['g']), _col(d1['beta']),
        d2['w'], _col(d2['b']), _col(d2['g']), _col(d2['beta']),
        d3['w'], _col(d3['b']), _col(d3['g']), _col(d3['beta']),
        p['linear']['w'], _col(p['linear']['b']),
        _col(p['bn_linear']['g']), _col(p['bn_linear']['beta']),
    ]


def _w_specs(args, start):
    return [pl.BlockSpec(a.shape, lambda b: (0,) * a.ndim) for a in args[start:]]


def _sa_layer(p, xyzT, ptsT, npoint, nsample, bw, interpret=False, new_xyz=None):
    B, _, N = xyzT.shape
    C = ptsT.shape[1]
    S, K = npoint, nsample
    O = p['mlp'][0]['w'].shape[0]
    Cout = p['linear']['w'].shape[0]
    Ct = C + 4

    nxT = _fps_new_xyz(xyzT, S, interpret=interpret) if new_xyz is None else new_xyz
    xyzN = jnp.transpose(xyzT, (0, 2, 1))
    wargs = _sa_weight_args(p, O, Cout)

    in_specs = [
        pl.BlockSpec((1, N, 3), lambda b: (b, 0, 0)),
        pl.BlockSpec((1, 3, N), lambda b: (b, 0, 0)),
        pl.BlockSpec((1, C, N), lambda b: (b, 0, 0)),
        pl.BlockSpec((1, 3, S), lambda b: (b, 0, 0)),
        pl.BlockSpec((1, S, 3), lambda b: (b, 0, 0)),
    ] + [pl.BlockSpec(a.shape, functools.partial(lambda nd, b: (0,) * nd, a.ndim))
         for a in wargs]

    out = pl.pallas_call(
        functools.partial(_sa_body_kernel, N, S, K, C, O, Cout, bw),
        grid=(B,),
        in_specs=in_specs,
        out_specs=pl.BlockSpec((1, Cout, S), lambda b: (b, 0, 0)),
        out_shape=jax.ShapeDtypeStruct((B, Cout, S), _F32),
        scratch_shapes=[
            pltpu.VMEM((N, S), _F32),
            pltpu.VMEM((K, Ct, S), _F32),
            pltpu.VMEM((K, O, S), _F32),
            pltpu.VMEM((K, 8, S), _F32),
            pltpu.VMEM((K, 8, S), _F32),
            pltpu.VMEM((K, 16, S), _F32),
            pltpu.VMEM((K, 16, S), _F32),
            pltpu.VMEM((K, 8, S), _F32),
            pltpu.VMEM((K, 1, S), _F32),
            pltpu.VMEM((O, 16, S), _F32),
        ],
        compiler_params=pltpu.CompilerParams(dimension_semantics=("parallel",)),
        interpret=interpret,
    )(xyzN, xyzT, ptsT, nxT, jnp.transpose(nxT, (0, 2, 1)), *wargs)
    return nxT, out


def _sa_all_kernel(N, C, O, Cout, bw,
                   xyzN_ref, xyzT_ref, ptsT_ref,
                   wm_ref, bm_ref, gm_ref, bem_ref,
                   ww1_ref, bw1_ref, gw1_ref, bew1_ref,
                   ww2_ref, bw2_ref, gw2_ref, bew2_ref,
                   ww3_ref, bw3_ref, gw3_ref, bew3_ref,
                   wd1_ref, bd1_ref, gd1_ref, bed1_ref,
                   wd2_ref, bd2_ref, gd2_ref, bed2_ref,
                   wd3_ref, bd3_ref, gd3_ref, bed3_ref,
                   lw3_ref, lb_ref, gl_ref, bel_ref,
                   out_ref):
    xyzN = xyzN_ref[0]
    X = xyzT_ref[0]
    P = ptsT_ref[0]

    a_row = jnp.sum(X * X, axis=0, keepdims=True)
    a_col = jnp.sum(xyzN * xyzN, axis=1, keepdims=True)
    Gm = jnp.dot(xyzN, X, preferred_element_type=_F32)
    sqr_full = (a_col + a_row) - 2.0 * Gm
    gk = jnp.exp(-sqr_full / (2.0 * bw * bw)) / (2.5 * bw)
    invd = N / jnp.sum(gk, axis=0, keepdims=True)

    def gn_full(x, cnt, g, beta):
        mu = jnp.sum(x, axis=(0, 1), keepdims=True)[0] / cnt
        xc = x - mu
        var = jnp.sum(xc * xc, axis=(0, 1), keepdims=True)[0] / cnt
        return xc / jnp.sqrt(var + 1e-5) * g + beta

    npts = jnp.concatenate([X, P], axis=0)
    h = jnp.dot(wm_ref[...], npts, preferred_element_type=_F32) + bm_ref[...]
    h = jnp.maximum(gn_full(h, float(O * N), gm_ref[...], bem_ref[...]), 0.0)

    dsc = invd / jnp.max(invd, axis=1, keepdims=True)
    d1 = jnp.maximum(gn_full(wd1_ref[...] * dsc + bd1_ref[...], float(16 * N),
                             gd1_ref[...], bed1_ref[...]), 0.0)
    d2 = jnp.dot(wd2_ref[...], d1, preferred_element_type=_F32) + bd2_ref[...]
    d2 = jnp.maximum(gn_full(d2, float(8 * N), gd2_ref[...], bed2_ref[...]), 0.0)
    d3 = jnp.dot(wd3_ref[...], d2, preferred_element_type=_F32) + bd3_ref[...]
    d3 = _sigmoid(gn_full(d3, float(N), gd3_ref[...], bed3_ref[...]))

    x = h * d3

    w1 = jnp.dot(ww1_ref[...], X, preferred_element_type=_F32) + bw1_ref[...]
    w1 = jnp.maximum(gn_full(w1, float(8 * N), gw1_ref[...], bew1_ref[...]), 0.0)
    w2 = jnp.dot(ww2_ref[...], w1, preferred_element_type=_F32) + bw2_ref[...]
    w2 = jnp.maximum(gn_full(w2, float(8 * N), gw2_ref[...], bew2_ref[...]), 0.0)
    w3 = jnp.dot(ww3_ref[...], w2, preferred_element_type=_F32) + bw3_ref[...]
    wt = jnp.maximum(gn_full(w3, float(16 * N), gw3_ref[...], bew3_ref[...]), 0.0)

    y = jnp.zeros((Cout, N), _F32)
    for w in range(16):
        tw = jnp.dot(lw3_ref[w], x, preferred_element_type=_F32)
        y = y + tw * wt[w:w + 1, :]
    lout = jnp.sum(y, axis=1, keepdims=True) + lb_ref[...]

    mu = jnp.sum(lout, axis=(0, 1), keepdims=True)[0] / float(Cout)
    xc = lout - mu
    var = jnp.sum(xc * xc, axis=(0, 1), keepdims=True)[0] / float(Cout)
    out_ref[0] = jnp.maximum(xc / jnp.sqrt(var + 1e-5) * gl_ref[...] + bel_ref[...], 0.0)


def _sa_layer_all(p, xyzT, ptsT, bw, interpret=False):
    B, _, N = xyzT.shape
    C = ptsT.shape[1]
    O = p['mlp'][0]['w'].shape[0]
    Cout = p['linear']['w'].shape[0]

    wargs = _sa_weight_args(p, O, Cout)
    wargs[28] = jnp.transpose(wargs[28].reshape(Cout, O, 16), (2, 0, 1))
    xyzN = jnp.transpose(xyzT, (0, 2, 1))

    in_specs = [
        pl.BlockSpec((1, N, 3), lambda b: (b, 0, 0)),
        pl.BlockSpec((1, 3, N), lambda b: (b, 0, 0)),
        pl.BlockSpec((1, C, N), lambda b: (b, 0, 0)),
    ] + [pl.BlockSpec(a.shape, functools.partial(lambda nd, b: (0,) * nd, a.ndim))
         for a in wargs]

    out = pl.pallas_call(
        functools.partial(_sa_all_kernel, N, C, O, Cout, bw),
        grid=(B,),
        in_specs=in_specs,
        out_specs=pl.BlockSpec((1, Cout, 1), lambda b: (b, 0, 0)),
        out_shape=jax.ShapeDtypeStruct((B, Cout, 1), _F32),
        compiler_params=pltpu.CompilerParams(dimension_semantics=("parallel",)),
        interpret=interpret,
    )(xyzN, xyzT, ptsT, *wargs)
    return out[:, :, 0]


def _fc_head_kernel(h_ref, w1_ref, b1_ref, g1_ref, be1_ref,
                    w3_ref, b3_ref, g3_ref, be3_ref,
                    w4_ref, b4_ref, g4_ref, be4_ref,
                    w5_ref, b5_ref, out_ref):
    def gn_relu(x, g, b, eps=1e-5):
        mu = jnp.mean(x, axis=1, keepdims=True)
        var = jnp.mean((x - mu) ** 2, axis=1, keepdims=True)
        xh = (x - mu) / jnp.sqrt(var + eps)
        return jnp.maximum(xh * g[None, :] + b[None, :], 0.0)

    h = h_ref[...]
    h = gn_relu(jnp.dot(h, w1_ref[...].T, preferred_element_type=_F32) + b1_ref[...][None, :],
                g1_ref[...], be1_ref[...])
    h = gn_relu(jnp.dot(h, w3_ref[...].T, preferred_element_type=_F32) + b3_ref[...][None, :],
                g3_ref[...], be3_ref[...])
    h = gn_relu(jnp.dot(h, w4_ref[...].T, preferred_element_type=_F32) + b4_ref[...][None, :],
                g4_ref[...], be4_ref[...])
    out_ref[...] = jnp.dot(h, w5_ref[...].T, preferred_element_type=_F32) + b5_ref[...][None, :]


def _fc_head(params, h, interpret=False):
    B = h.shape[0]
    args = [h,
            params['fc1']['w'], params['fc1']['b'], params['bn1']['g'], params['bn1']['beta'],
            params['fc3']['w'], params['fc3']['b'], params['bn3']['g'], params['bn3']['beta'],
            params['fc4']['w'], params['fc4']['b'], params['bn4']['g'], params['bn4']['beta'],
            params['fc5']['w'], params['fc5']['b']]
    return pl.pallas_call(
        _fc_head_kernel,
        out_shape=jax.ShapeDtypeStruct((B, 6), jnp.float32),
        interpret=interpret,
    )(*args)


def _fps_pair(xyzT_a, xyzT_b, npoint, interpret=False):
    B = xyzT_a.shape[0]
    nx = _fps_new_xyz(jnp.concatenate([xyzT_a, xyzT_b], axis=0), npoint, interpret)
    return nx[:B], nx[B:]


def _forward(params, xyz, xyz_goal, interpret=False):
    nx1, ng1 = _fps_pair(xyz[:, :3, :], xyz_goal, 512, interpret)
    l1x, l1p = _sa_layer(params['sa1'], xyz[:, :3, :], xyz, 512, 32, 0.1, interpret,
                         new_xyz=nx1)
    g1x, g1p = _sa_layer(params['sa1_g'], xyz_goal, xyz_goal, 512, 32, 0.1, interpret,
                         new_xyz=ng1)
    nx2, ng2 = _fps_pair(l1x, g1x, 128, interpret)
    l2x, l2p = _sa_layer(params['sa2'], l1x, l1p, 128, 64, 0.2, interpret, new_xyz=nx2)
    g2x, g2p = _sa_layer(params['sa2_g'], g1x, g1p, 128, 64, 0.2, interpret, new_xyz=ng2)
    x = _sa_layer_all(params['sa3'], l2x, l2p, 0.4, interpret)
    g = _sa_layer_all(params['sa3_g'], g2x, g2p, 0.4, interpret)
    h = jnp.concatenate([x, g], axis=-1)
    return _fc_head(params, h, interpret)


def kernel(xyz, xyz_goal, params):
    return _forward(params, xyz, xyz_goal)


# speedup vs baseline: 1.0070x; 1.0070x over previous
import functools

import jax
import jax.numpy as jnp
from jax import lax
from jax.experimental import pallas as pl
from jax.experimental.pallas import tpu as pltpu

_F32 = jnp.float32


def _fps_kernel(xyz_ref, ox_ref, oy_ref, oz_ref):
    _, B, N = xyz_ref.shape
    S = ox_ref.shape[1]
    col = lax.broadcasted_iota(jnp.int32, (B, N), 1)
    colS = lax.broadcasted_iota(jnp.int32, (B, S), 1)
    X = xyz_ref[0]
    Y = xyz_ref[1]
    Z = xyz_ref[2]

    def step(t, carry):
        dist, far = carry
        onehot = col == far
        cx = jnp.sum(jnp.where(onehot, X, 0.0), axis=1, keepdims=True)
        cy = jnp.sum(jnp.where(onehot, Y, 0.0), axis=1, keepdims=True)
        cz = jnp.sum(jnp.where(onehot, Z, 0.0), axis=1, keepdims=True)
        emit = colS == t
        ox_ref[...] = jnp.where(emit, cx, ox_ref[...])
        oy_ref[...] = jnp.where(emit, cy, oy_ref[...])
        oz_ref[...] = jnp.where(emit, cz, oz_ref[...])
        dx = X - cx
        dy = Y - cy
        dz = Z - cz
        dist = jnp.minimum(dist, (dx * dx + dy * dy) + dz * dz)
        dmax = jnp.max(dist, axis=1, keepdims=True)
        far = jnp.min(jnp.where(dist == dmax, col, N), axis=1, keepdims=True)
        return dist, far

    lax.fori_loop(0, S, step,
                  (jnp.full((B, N), 1e10, _F32), jnp.zeros((B, 1), jnp.int32)))


def _fps_new_xyz(xyzT, npoint, interpret=False):
    B, _, N = xyzT.shape
    shp = jax.ShapeDtypeStruct((B, npoint), _F32)
    ox, oy, oz = pl.pallas_call(
        _fps_kernel,
        out_shape=(shp, shp, shp),
        interpret=interpret,
    )(jnp.transpose(xyzT, (1, 0, 2)))
    return jnp.stack([ox, oy, oz], axis=1)


def _sum11(x):
    return jnp.sum(jnp.sum(x, axis=1, keepdims=True), axis=0, keepdims=True)


def _gn_apply(x, mu, var, g, beta, eps=1e-5):
    return (x - mu) / jnp.sqrt(var + eps) * g + beta


def _sigmoid(x):
    return 1.0 / (1.0 + jnp.exp(-x))


def _sa_body_kernel(N, S, K, C, O, Cout, bw,
                    xyzN_ref, xyzT_ref, ptsT_ref, nxT_ref,
                    wm_ref, bm_ref, gm_ref, bem_ref,
                    ww1_ref, bw1_ref, gw1_ref, bew1_ref,
                    ww2_ref, bw2_ref, gw2_ref, bew2_ref,
                    ww3_ref, bw3_ref, gw3_ref, bew3_ref,
                    wd1_ref, bd1_ref, gd1_ref, bed1_ref,
                    wd2_ref, bd2_ref, gd2_ref, bed2_ref,
                    wd3_ref, bd3_ref, gd3_ref, bed3_ref,
                    lw_ref, lb_ref, gl_ref, bel_ref,
                    out_ref,
                    sqr_ref, buf_ref, hbuf_ref, w1b_ref, w2b_ref, w3b_ref,
                    d1b_ref, d2b_ref, d3b_ref, acc_ref):
    Ct = C + 4
    xyzN = xyzN_ref[0]
    X = xyzT_ref[0]
    P = ptsT_ref[0]
    NX = nxT_ref[0]

    a_row = jnp.sum(X * X, axis=0, keepdims=True)
    a_col = jnp.sum(xyzN * xyzN, axis=1, keepdims=True)
    Gm = jnp.dot(xyzN, X, preferred_element_type=_F32)
    sqr_full = (a_col + a_row) - 2.0 * Gm
    gk = jnp.exp(-sqr_full / (2.0 * bw * bw)) / (2.5 * bw)
    invd = N / jnp.sum(gk, axis=0, keepdims=True)

    s_row = jnp.sum(NX * NX, axis=0, keepdims=True)
    Gn = jnp.dot(xyzN, NX, preferred_element_type=_F32)
    sqr_ref[...] = (a_col + s_row) - 2.0 * Gn

    TT = jnp.concatenate([X, P, invd], axis=0)
    rowN = lax.broadcasted_iota(jnp.int32, (N, S), 0)

    def gather_step(j, gdmax):
        sq = sqr_ref[...]
        m = jnp.min(sq, axis=0, keepdims=True)
        first = jnp.min(jnp.where(sq == m, rowN, N), axis=0, keepdims=True)
        selb = rowN == first
        Gj = jnp.dot(TT, selb.astype(_F32), preferred_element_type=_F32)
        buf_ref[pl.ds(j, 1)] = Gj[None]
        sqr_ref[...] = jnp.where(selb, 1e30, sq)
        return jnp.maximum(gdmax, Gj[Ct - 1:Ct, :])

    gdmax = lax.fori_loop(0, K, gather_step, jnp.full((1, S), -1e30, _F32), unroll=4)

    wm = wm_ref[...]
    bm = bm_ref[...]
    ww1 = ww1_ref[...]
    bw1 = bw1_ref[...]
    wd1 = wd1_ref[...]
    bd1 = bd1_ref[...]

    z = jnp.zeros((1, 1), _F32)

    def loopA(j, cr):
        sh, sh2, sw, sw2, sd, sd2 = cr
        Gj = buf_ref[pl.ds(j, 1)][0]
        xn = Gj[0:3, :] - NX
        npj = jnp.concatenate([xn, Gj[3:3 + C, :]], axis=0)
        hj = jnp.dot(wm, npj, preferred_element_type=_F32) + bm
        hbuf_ref[pl.ds(j, 1)] = hj[None]
        w1j = jnp.dot(ww1, xn, preferred_element_type=_F32) + bw1
        w1b_ref[pl.ds(j, 1)] = w1j[None]
        dscj = Gj[Ct - 1:Ct, :] / gdmax
        d1j = wd1 * dscj + bd1
        d1b_ref[pl.ds(j, 1)] = d1j[None]
        return (sh + _sum11(hj), sh2 + _sum11(hj * hj),
                sw + _sum11(w1j), sw2 + _sum11(w1j * w1j),
                sd + _sum11(d1j), sd2 + _sum11(d1j * d1j))

    sh, sh2, sw, sw2, sd, sd2 = lax.fori_loop(0, K, loopA, (z, z, z, z, z, z), unroll=8)
    ch = float(O * K * S)
    mu_h = sh / ch
    var_h = sh2 / ch - mu_h * mu_h
    cw1 = float(8 * K * S)
    mu_w1 = sw / cw1
    var_w1 = sw2 / cw1 - mu_w1 * mu_w1
    cd1 = float(16 * K * S)
    mu_d1 = sd / cd1
    var_d1 = sd2 / cd1 - mu_d1 * mu_d1

    ww2 = ww2_ref[...]
    bw2 = bw2_ref[...]
    wd2 = wd2_ref[...]
    bd2 = bd2_ref[...]
    gw1 = gw1_ref[...]
    bew1 = bew1_ref[...]
    gd1 = gd1_ref[...]
    bed1 = bed1_ref[...]

    def loopB(j, cr):
        sw_, sw2_, sd_, sd2_ = cr
        w1j = jnp.maximum(_gn_apply(w1b_ref[pl.ds(j, 1)][0], mu_w1, var_w1, gw1, bew1), 0.0)
        w2j = jnp.dot(ww2, w1j, preferred_element_type=_F32) + bw2
        w2b_ref[pl.ds(j, 1)] = w2j[None]
        d1j = jnp.maximum(_gn_apply(d1b_ref[pl.ds(j, 1)][0], mu_d1, var_d1, gd1, bed1), 0.0)
        d2j = jnp.dot(wd2, d1j, preferred_element_type=_F32) + bd2
        d2b_ref[pl.ds(j, 1)] = d2j[None]
        return (sw_ + _sum11(w2j), sw2_ + _sum11(w2j * w2j),
                sd_ + _sum11(d2j), sd2_ + _sum11(d2j * d2j))

    sw, sw2, sd, sd2 = lax.fori_loop(0, K, loopB, (z, z, z, z), unroll=8)
    cw2 = float(8 * K * S)
    mu_w2 = sw / cw2
    var_w2 = sw2 / cw2 - mu_w2 * mu_w2
    cd2 = float(8 * K * S)
    mu_d2 = sd / cd2
    var_d2 = sd2 / cd2 - mu_d2 * mu_d2

    ww3 = ww3_ref[...]
    bw3 = bw3_ref[...]
    wd3 = wd3_ref[...]
    bd3 = bd3_ref[...]
    gw2 = gw2_ref[...]
    bew2 = bew2_ref[...]
    gd2 = gd2_ref[...]
    bed2 = bed2_ref[...]

    def loopC(j, cr):
        sw_, sw2_, sd_, sd2_ = cr
        w2j = jnp.maximum(_gn_apply(w2b_ref[pl.ds(j, 1)][0], mu_w2, var_w2, gw2, bew2), 0.0)
        w3j = jnp.dot(ww3, w2j, preferred_element_type=_F32) + bw3
        w3b_ref[pl.ds(j, 1)] = w3j[None]
        d2j = jnp.maximum(_gn_apply(d2b_ref[pl.ds(j, 1)][0], mu_d2, var_d2, gd2, bed2), 0.0)
        d3j = jnp.dot(wd3, d2j, preferred_element_type=_F32) + bd3
        d3b_ref[pl.ds(j, 1)] = d3j[None]
        return (sw_ + _sum11(w3j), sw2_ + _sum11(w3j * w3j),
                sd_ + _sum11(d3j), sd2_ + _sum11(d3j * d3j))

    sw, sw2, sd, sd2 = lax.fori_loop(0, K, loopC, (z, z, z, z), unroll=8)
    cw3 = float(16 * K * S)
    mu_w3 = sw / cw3
    var_w3 = sw2 / cw3 - mu_w3 * mu_w3
    cd3 = float(1 * K * S)
    mu_d3 = sd / cd3
    var_d3 = sd2 / cd3 - mu_d3 * mu_d3

    gm = gm_ref[...]
    bem = bem_ref[...]
    gw3 = gw3_ref[...]
    bew3 = bew3_ref[...]
    gd3 = gd3_ref[...]
    bed3 = bed3_ref[...]

    acc_ref[...] = jnp.zeros((O, 16, S), _F32)

    def loopD(j, _):
        hj = jnp.maximum(_gn_apply(hbuf_ref[pl.ds(j, 1)][0], mu_h, var_h, gm, bem), 0.0)
        d3j = _sigmoid(_gn_apply(d3b_ref[pl.ds(j, 1)][0], mu_d3, var_d3, gd3, bed3))
        xj = hj * d3j
        wtj = jnp.maximum(_gn_apply(w3b_ref[pl.ds(j, 1)][0], mu_w3, var_w3, gw3, bew3), 0.0)
        acc_ref[...] += xj[:, None, :] * wtj[None, :, :]
        return 0

    lax.fori_loop(0, K, loopD, 0, unroll=8)

    flat = acc_ref[...].reshape(O * 16, S)
    lout = jnp.dot(lw_ref[...], flat, preferred_element_type=_F32) + lb_ref[...]
    mu = jnp.sum(lout, axis=(0, 1), keepdims=True)[0] / float(Cout * S)
    xc = lout - mu
    var = jnp.sum(xc * xc, axis=(0, 1), keepdims=True)[0] / float(Cout * S)
    out_ref[0] = jnp.maximum(xc / jnp.sqrt(var + 1e-5) * gl_ref[...] + bel_ref[...], 0.0)


def _col(v):
    return v[:, None]


def _sa_weight_args(p, O, Cout):
    m = p['mlp'][0]
    w1, w2, w3 = p['weight']
    d1, d2, d3 = p['density']
    return [
        m['w'], _col(m['b']), _col(m['g']), _col(m['beta']),
        w1['w'], _col(w1['b']), _col(w1['g']), _col(w1['beta']),
        w2['w'], _col(w2['b']), _col(w2['g']), _col(w2['beta']),
        w3['w'], _col(w3['b']), _col(w3['g']), _col(w3['beta']),
        d1['w'], _col(d1['b']), _col(d1['g']), _col(d1['beta']),
        d2['w'], _col(d2['b']), _col(d2['g']), _col(d2['beta']),
        d3['w'], _col(d3['b']), _col(d3['g']), _col(d3['beta']),
        p['linear']['w'], _col(p['linear']['b']),
        _col(p['bn_linear']['g']), _col(p['bn_linear']['beta']),
    ]


def _w_specs(args, start):
    return [pl.BlockSpec(a.shape, lambda b: (0,) * a.ndim) for a in args[start:]]


def _sa_layer(p, xyzT, ptsT, npoint, nsample, bw, interpret=False, new_xyz=None):
    B, _, N = xyzT.shape
    C = ptsT.shape[1]
    S, K = npoint, nsample
    O = p['mlp'][0]['w'].shape[0]
    Cout = p['linear']['w'].shape[0]
    Ct = C + 4

    nxT = _fps_new_xyz(xyzT, S, interpret=interpret) if new_xyz is None else new_xyz
    xyzN = jnp.transpose(xyzT, (0, 2, 1))
    wargs = _sa_weight_args(p, O, Cout)

    in_specs = [
        pl.BlockSpec((1, N, 3), lambda b: (b, 0, 0)),
        pl.BlockSpec((1, 3, N), lambda b: (b, 0, 0)),
        pl.BlockSpec((1, C, N), lambda b: (b, 0, 0)),
        pl.BlockSpec((1, 3, S), lambda b: (b, 0, 0)),
    ] + [pl.BlockSpec(a.shape, functools.partial(lambda nd, b: (0,) * nd, a.ndim))
         for a in wargs]

    out = pl.pallas_call(
        functools.partial(_sa_body_kernel, N, S, K, C, O, Cout, bw),
        grid=(B,),
        in_specs=in_specs,
        out_specs=pl.BlockSpec((1, Cout, S), lambda b: (b, 0, 0)),
        out_shape=jax.ShapeDtypeStruct((B, Cout, S), _F32),
        scratch_shapes=[
            pltpu.VMEM((N, S), _F32),
            pltpu.VMEM((K, Ct, S), _F32),
            pltpu.VMEM((K, O, S), _F32),
            pltpu.VMEM((K, 8, S), _F32),
            pltpu.VMEM((K, 8, S), _F32),
            pltpu.VMEM((K, 16, S), _F32),
            pltpu.VMEM((K, 16, S), _F32),
            pltpu.VMEM((K, 8, S), _F32),
            pltpu.VMEM((K, 1, S), _F32),
            pltpu.VMEM((O, 16, S), _F32),
        ],
        compiler_params=pltpu.CompilerParams(dimension_semantics=("parallel",)),
        interpret=interpret,
    )(xyzN, xyzT, ptsT, nxT, *wargs)
    return nxT, out


def _sa_all_kernel(N, C, O, Cout, bw,
                   xyzN_ref, xyzT_ref, ptsT_ref,
                   wm_ref, bm_ref, gm_ref, bem_ref,
                   ww1_ref, bw1_ref, gw1_ref, bew1_ref,
                   ww2_ref, bw2_ref, gw2_ref, bew2_ref,
                   ww3_ref, bw3_ref, gw3_ref, bew3_ref,
                   wd1_ref, bd1_ref, gd1_ref, bed1_ref,
                   wd2_ref, bd2_ref, gd2_ref, bed2_ref,
                   wd3_ref, bd3_ref, gd3_ref, bed3_ref,
                   lw3_ref, lb_ref, gl_ref, bel_ref,
                   out_ref):
    xyzN = xyzN_ref[0]
    X = xyzT_ref[0]
    P = ptsT_ref[0]

    a_row = jnp.sum(X * X, axis=0, keepdims=True)
    a_col = jnp.sum(xyzN * xyzN, axis=1, keepdims=True)
    Gm = jnp.dot(xyzN, X, preferred_element_type=_F32)
    sqr_full = (a_col + a_row) - 2.0 * Gm
    gk = jnp.exp(-sqr_full / (2.0 * bw * bw)) / (2.5 * bw)
    invd = N / jnp.sum(gk, axis=0, keepdims=True)

    def gn_full(x, cnt, g, beta):
        mu = jnp.sum(x, axis=(0, 1), keepdims=True)[0] / cnt
        xc = x - mu
        var = jnp.sum(xc * xc, axis=(0, 1), keepdims=True)[0] / cnt
        return xc / jnp.sqrt(var + 1e-5) * g + beta

    npts = jnp.concatenate([X, P], axis=0)
    h = jnp.dot(wm_ref[...], npts, preferred_element_type=_F32) + bm_ref[...]
    h = jnp.maximum(gn_full(h, float(O * N), gm_ref[...], bem_ref[...]), 0.0)

    dsc = invd / jnp.max(invd, axis=1, keepdims=True)
    d1 = jnp.maximum(gn_full(wd1_ref[...] * dsc + bd1_ref[...], float(16 * N),
                             gd1_ref[...], bed1_ref[...]), 0.0)
    d2 = jnp.dot(wd2_ref[...], d1, preferred_element_type=_F32) + bd2_ref[...]
    d2 = jnp.maximum(gn_full(d2, float(8 * N), gd2_ref[...], bed2_ref[...]), 0.0)
    d3 = jnp.dot(wd3_ref[...], d2, preferred_element_type=_F32) + bd3_ref[...]
    d3 = _sigmoid(gn_full(d3, float(N), gd3_ref[...], bed3_ref[...]))

    x = h * d3

    w1 = jnp.dot(ww1_ref[...], X, preferred_element_type=_F32) + bw1_ref[...]
    w1 = jnp.maximum(gn_full(w1, float(8 * N), gw1_ref[...], bew1_ref[...]), 0.0)
    w2 = jnp.dot(ww2_ref[...], w1, preferred_element_type=_F32) + bw2_ref[...]
    w2 = jnp.maximum(gn_full(w2, float(8 * N), gw2_ref[...], bew2_ref[...]), 0.0)
    w3 = jnp.dot(ww3_ref[...], w2, preferred_element_type=_F32) + bw3_ref[...]
    wt = jnp.maximum(gn_full(w3, float(16 * N), gw3_ref[...], bew3_ref[...]), 0.0)

    y = jnp.zeros((Cout, N), _F32)
    for w in range(16):
        tw = jnp.dot(lw3_ref[w], x, preferred_element_type=_F32)
        y = y + tw * wt[w:w + 1, :]
    lout = jnp.sum(y, axis=1, keepdims=True) + lb_ref[...]

    mu = jnp.sum(lout, axis=(0, 1), keepdims=True)[0] / float(Cout)
    xc = lout - mu
    var = jnp.sum(xc * xc, axis=(0, 1), keepdims=True)[0] / float(Cout)
    out_ref[0] = jnp.maximum(xc / jnp.sqrt(var + 1e-5) * gl_ref[...] + bel_ref[...], 0.0)


def _sa_layer_all(p, xyzT, ptsT, bw, interpret=False):
    B, _, N = xyzT.shape
    C = ptsT.shape[1]
    O = p['mlp'][0]['w'].shape[0]
    Cout = p['linear']['w'].shape[0]

    wargs = _sa_weight_args(p, O, Cout)
    wargs[28] = jnp.transpose(wargs[28].reshape(Cout, O, 16), (2, 0, 1))
    xyzN = jnp.transpose(xyzT, (0, 2, 1))

    in_specs = [
        pl.BlockSpec((1, N, 3), lambda b: (b, 0, 0)),
        pl.BlockSpec((1, 3, N), lambda b: (b, 0, 0)),
        pl.BlockSpec((1, C, N), lambda b: (b, 0, 0)),
    ] + [pl.BlockSpec(a.shape, functools.partial(lambda nd, b: (0,) * nd, a.ndim))
         for a in wargs]

    out = pl.pallas_call(
        functools.partial(_sa_all_kernel, N, C, O, Cout, bw),
        grid=(B,),
        in_specs=in_specs,
        out_specs=pl.BlockSpec((1, Cout, 1), lambda b: (b, 0, 0)),
        out_shape=jax.ShapeDtypeStruct((B, Cout, 1), _F32),
        compiler_params=pltpu.CompilerParams(dimension_semantics=("parallel",)),
        interpret=interpret,
    )(xyzN, xyzT, ptsT, *wargs)
    return out[:, :, 0]


def _fc_head_kernel(h_ref, w1_ref, b1_ref, g1_ref, be1_ref,
                    w3_ref, b3_ref, g3_ref, be3_ref,
                    w4_ref, b4_ref, g4_ref, be4_ref,
                    w5_ref, b5_ref, out_ref):
    def gn_relu(x, g, b, eps=1e-5):
        mu = jnp.mean(x, axis=1, keepdims=True)
        var = jnp.mean((x - mu) ** 2, axis=1, keepdims=True)
        xh = (x - mu) / jnp.sqrt(var + eps)
        return jnp.maximum(xh * g[None, :] + b[None, :], 0.0)

    h = h_ref[...]
    h = gn_relu(jnp.dot(h, w1_ref[...].T, preferred_element_type=_F32) + b1_ref[...][None, :],
                g1_ref[...], be1_ref[...])
    h = gn_relu(jnp.dot(h, w3_ref[...].T, preferred_element_type=_F32) + b3_ref[...][None, :],
                g3_ref[...], be3_ref[...])
    h = gn_relu(jnp.dot(h, w4_ref[...].T, preferred_element_type=_F32) + b4_ref[...][None, :],
                g4_ref[...], be4_ref[...])
    out_ref[...] = jnp.dot(h, w5_ref[...].T, preferred_element_type=_F32) + b5_ref[...][None, :]


def _fc_head(params, h, interpret=False):
    B = h.shape[0]
    args = [h,
            params['fc1']['w'], params['fc1']['b'], params['bn1']['g'], params['bn1']['beta'],
            params['fc3']['w'], params['fc3']['b'], params['bn3']['g'], params['bn3']['beta'],
            params['fc4']['w'], params['fc4']['b'], params['bn4']['g'], params['bn4']['beta'],
            params['fc5']['w'], params['fc5']['b']]
    return pl.pallas_call(
        _fc_head_kernel,
        out_shape=jax.ShapeDtypeStruct((B, 6), jnp.float32),
        interpret=interpret,
    )(*args)


def _fps_pair(xyzT_a, xyzT_b, npoint, interpret=False):
    B = xyzT_a.shape[0]
    nx = _fps_new_xyz(jnp.concatenate([xyzT_a, xyzT_b], axis=0), npoint, interpret)
    return nx[:B], nx[B:]


def _forward(params, xyz, xyz_goal, interpret=False):
    nx1, ng1 = _fps_pair(xyz[:, :3, :], xyz_goal, 512, interpret)
    l1x, l1p = _sa_layer(params['sa1'], xyz[:, :3, :], xyz, 512, 32, 0.1, interpret,
                         new_xyz=nx1)
    g1x, g1p = _sa_layer(params['sa1_g'], xyz_goal, xyz_goal, 512, 32, 0.1, interpret,
                         new_xyz=ng1)
    nx2, ng2 = _fps_pair(l1x, g1x, 128, interpret)
    l2x, l2p = _sa_layer(params['sa2'], l1x, l1p, 128, 64, 0.2, interpret, new_xyz=nx2)
    g2x, g2p = _sa_layer(params['sa2_g'], g1x, g1p, 128, 64, 0.2, interpret, new_xyz=ng2)
    x = _sa_layer_all(params['sa3'], l2x, l2p, 0.4, interpret)
    g = _sa_layer_all(params['sa3_g'], g2x, g2p, 0.4, interpret)
    h = jnp.concatenate([x, g], axis=-1)
    return _fc_head(params, h, interpret)


def kernel(xyz, xyz_goal, params):
    return _forward(params, xyz, xyz_goal)


# speedup vs baseline: 1.0076x; 1.0006x over previous
import functools

import jax
import jax.numpy as jnp
from jax import lax
from jax.experimental import pallas as pl
from jax.experimental.pallas import tpu as pltpu

_F32 = jnp.float32


def _fps_kernel(xyz_ref, ox_ref, oy_ref, oz_ref):
    _, B, N = xyz_ref.shape
    S = ox_ref.shape[1]
    col = lax.broadcasted_iota(jnp.int32, (B, N), 1)
    colS = lax.broadcasted_iota(jnp.int32, (B, S), 1)
    X = xyz_ref[0]
    Y = xyz_ref[1]
    Z = xyz_ref[2]

    def step(t, carry):
        dist, far = carry
        onehot = col == far
        cx = jnp.sum(jnp.where(onehot, X, 0.0), axis=1, keepdims=True)
        cy = jnp.sum(jnp.where(onehot, Y, 0.0), axis=1, keepdims=True)
        cz = jnp.sum(jnp.where(onehot, Z, 0.0), axis=1, keepdims=True)
        emit = colS == t
        ox_ref[...] = jnp.where(emit, cx, ox_ref[...])
        oy_ref[...] = jnp.where(emit, cy, oy_ref[...])
        oz_ref[...] = jnp.where(emit, cz, oz_ref[...])
        dx = X - cx
        dy = Y - cy
        dz = Z - cz
        dist = jnp.minimum(dist, (dx * dx + dy * dy) + dz * dz)
        dmax = jnp.max(dist, axis=1, keepdims=True)
        far = jnp.min(jnp.where(dist == dmax, col, N), axis=1, keepdims=True)
        return dist, far

    lax.fori_loop(0, S, step,
                  (jnp.full((B, N), 1e10, _F32), jnp.zeros((B, 1), jnp.int32)),
                  unroll=2)


def _fps_new_xyz(xyzT, npoint, interpret=False):
    B, _, N = xyzT.shape
    shp = jax.ShapeDtypeStruct((B, npoint), _F32)
    ox, oy, oz = pl.pallas_call(
        _fps_kernel,
        out_shape=(shp, shp, shp),
        interpret=interpret,
    )(jnp.transpose(xyzT, (1, 0, 2)))
    return jnp.stack([ox, oy, oz], axis=1)


def _sum11(x):
    return jnp.sum(jnp.sum(x, axis=1, keepdims=True), axis=0, keepdims=True)


def _gn_apply(x, mu, var, g, beta, eps=1e-5):
    return (x - mu) / jnp.sqrt(var + eps) * g + beta


def _sigmoid(x):
    return 1.0 / (1.0 + jnp.exp(-x))


def _sa_body_kernel(N, S, K, C, O, Cout, bw,
                    xyzN_ref, xyzT_ref, ptsT_ref, nxT_ref,
                    wm_ref, bm_ref, gm_ref, bem_ref,
                    ww1_ref, bw1_ref, gw1_ref, bew1_ref,
                    ww2_ref, bw2_ref, gw2_ref, bew2_ref,
                    ww3_ref, bw3_ref, gw3_ref, bew3_ref,
                    wd1_ref, bd1_ref, gd1_ref, bed1_ref,
                    wd2_ref, bd2_ref, gd2_ref, bed2_ref,
                    wd3_ref, bd3_ref, gd3_ref, bed3_ref,
                    lw_ref, lb_ref, gl_ref, bel_ref,
                    out_ref,
                    sqr_ref, buf_ref, hbuf_ref, w1b_ref, w2b_ref, w3b_ref,
                    d1b_ref, d2b_ref, d3b_ref, acc_ref):
    Ct = C + 4
    xyzN = xyzN_ref[0]
    X = xyzT_ref[0]
    P = ptsT_ref[0]
    NX = nxT_ref[0]

    a_row = jnp.sum(X * X, axis=0, keepdims=True)
    a_col = jnp.sum(xyzN * xyzN, axis=1, keepdims=True)
    Gm = jnp.dot(xyzN, X, preferred_element_type=_F32)
    sqr_full = (a_col + a_row) - 2.0 * Gm
    gk = jnp.exp(-sqr_full / (2.0 * bw * bw)) / (2.5 * bw)
    invd = N / jnp.sum(gk, axis=0, keepdims=True)

    s_row = jnp.sum(NX * NX, axis=0, keepdims=True)
    Gn = jnp.dot(xyzN, NX, preferred_element_type=_F32)
    sqr_ref[...] = (a_col + s_row) - 2.0 * Gn

    TT = jnp.concatenate([X, P, invd], axis=0)
    rowN = lax.broadcasted_iota(jnp.int32, (N, S), 0)

    def gather_step(j, gdmax):
        sq = sqr_ref[...]
        m = jnp.min(sq, axis=0, keepdims=True)
        first = jnp.min(jnp.where(sq == m, rowN, N), axis=0, keepdims=True)
        selb = rowN == first
        Gj = jnp.dot(TT, selb.astype(_F32), preferred_element_type=_F32)
        buf_ref[pl.ds(j, 1)] = Gj[None]
        sqr_ref[...] = jnp.where(selb, 1e30, sq)
        return jnp.maximum(gdmax, Gj[Ct - 1:Ct, :])

    gdmax = lax.fori_loop(0, K, gather_step, jnp.full((1, S), -1e30, _F32), unroll=4)

    wm = wm_ref[...]
    bm = bm_ref[...]
    ww1 = ww1_ref[...]
    bw1 = bw1_ref[...]
    wd1 = wd1_ref[...]
    bd1 = bd1_ref[...]

    z = jnp.zeros((1, 1), _F32)

    def loopA(j, cr):
        sh, sh2, sw, sw2, sd, sd2 = cr
        Gj = buf_ref[pl.ds(j, 1)][0]
        xn = Gj[0:3, :] - NX
        npj = jnp.concatenate([xn, Gj[3:3 + C, :]], axis=0)
        hj = jnp.dot(wm, npj, preferred_element_type=_F32) + bm
        hbuf_ref[pl.ds(j, 1)] = hj[None]
        w1j = jnp.dot(ww1, xn, preferred_element_type=_F32) + bw1
        w1b_ref[pl.ds(j, 1)] = w1j[None]
        dscj = Gj[Ct - 1:Ct, :] / gdmax
        d1j = wd1 * dscj + bd1
        d1b_ref[pl.ds(j, 1)] = d1j[None]
        return (sh + _sum11(hj), sh2 + _sum11(hj * hj),
                sw + _sum11(w1j), sw2 + _sum11(w1j * w1j),
                sd + _sum11(d1j), sd2 + _sum11(d1j * d1j))

    sh, sh2, sw, sw2, sd, sd2 = lax.fori_loop(0, K, loopA, (z, z, z, z, z, z), unroll=8)
    ch = float(O * K * S)
    mu_h = sh / ch
    var_h = sh2 / ch - mu_h * mu_h
    cw1 = float(8 * K * S)
    mu_w1 = sw / cw1
    var_w1 = sw2 / cw1 - mu_w1 * mu_w1
    cd1 = float(16 * K * S)
    mu_d1 = sd / cd1
    var_d1 = sd2 / cd1 - mu_d1 * mu_d1

    ww2 = ww2_ref[...]
    bw2 = bw2_ref[...]
    wd2 = wd2_ref[...]
    bd2 = bd2_ref[...]
    gw1 = gw1_ref[...]
    bew1 = bew1_ref[...]
    gd1 = gd1_ref[...]
    bed1 = bed1_ref[...]

    def loopB(j, cr):
        sw_, sw2_, sd_, sd2_ = cr
        w1j = jnp.maximum(_gn_apply(w1b_ref[pl.ds(j, 1)][0], mu_w1, var_w1, gw1, bew1), 0.0)
        w2j = jnp.dot(ww2, w1j, preferred_element_type=_F32) + bw2
        w2b_ref[pl.ds(j, 1)] = w2j[None]
        d1j = jnp.maximum(_gn_apply(d1b_ref[pl.ds(j, 1)][0], mu_d1, var_d1, gd1, bed1), 0.0)
        d2j = jnp.dot(wd2, d1j, preferred_element_type=_F32) + bd2
        d2b_ref[pl.ds(j, 1)] = d2j[None]
        return (sw_ + _sum11(w2j), sw2_ + _sum11(w2j * w2j),
                sd_ + _sum11(d2j), sd2_ + _sum11(d2j * d2j))

    sw, sw2, sd, sd2 = lax.fori_loop(0, K, loopB, (z, z, z, z), unroll=8)
    cw2 = float(8 * K * S)
    mu_w2 = sw / cw2
    var_w2 = sw2 / cw2 - mu_w2 * mu_w2
    cd2 = float(8 * K * S)
    mu_d2 = sd / cd2
    var_d2 = sd2 / cd2 - mu_d2 * mu_d2

    ww3 = ww3_ref[...]
    bw3 = bw3_ref[...]
    wd3 = wd3_ref[...]
    bd3 = bd3_ref[...]
    gw2 = gw2_ref[...]
    bew2 = bew2_ref[...]
    gd2 = gd2_ref[...]
    bed2 = bed2_ref[...]

    def loopC(j, cr):
        sw_, sw2_, sd_, sd2_ = cr
        w2j = jnp.maximum(_gn_apply(w2b_ref[pl.ds(j, 1)][0], mu_w2, var_w2, gw2, bew2), 0.0)
        w3j = jnp.dot(ww3, w2j, preferred_element_type=_F32) + bw3
        w3b_ref[pl.ds(j, 1)] = w3j[None]
        d2j = jnp.maximum(_gn_apply(d2b_ref[pl.ds(j, 1)][0], mu_d2, var_d2, gd2, bed2), 0.0)
        d3j = jnp.dot(wd3, d2j, preferred_element_type=_F32) + bd3
        d3b_ref[pl.ds(j, 1)] = d3j[None]
        return (sw_ + _sum11(w3j), sw2_ + _sum11(w3j * w3j),
                sd_ + _sum11(d3j), sd2_ + _sum11(d3j * d3j))

    sw, sw2, sd, sd2 = lax.fori_loop(0, K, loopC, (z, z, z, z), unroll=8)
    cw3 = float(16 * K * S)
    mu_w3 = sw / cw3
    var_w3 = sw2 / cw3 - mu_w3 * mu_w3
    cd3 = float(1 * K * S)
    mu_d3 = sd / cd3
    var_d3 = sd2 / cd3 - mu_d3 * mu_d3

    gm = gm_ref[...]
    bem = bem_ref[...]
    gw3 = gw3_ref[...]
    bew3 = bew3_ref[...]
    gd3 = gd3_ref[...]
    bed3 = bed3_ref[...]

    acc_ref[...] = jnp.zeros((O, 16, S), _F32)

    def loopD(j, _):
        hj = jnp.maximum(_gn_apply(hbuf_ref[pl.ds(j, 1)][0], mu_h, var_h, gm, bem), 0.0)
        d3j = _sigmoid(_gn_apply(d3b_ref[pl.ds(j, 1)][0], mu_d3, var_d3, gd3, bed3))
        xj = hj * d3j
        wtj = jnp.maximum(_gn_apply(w3b_ref[pl.ds(j, 1)][0], mu_w3, var_w3, gw3, bew3), 0.0)
        acc_ref[...] += xj[:, None, :] * wtj[None, :, :]
        return 0

    lax.fori_loop(0, K, loopD, 0, unroll=8)

    flat = acc_ref[...].reshape(O * 16, S)
    lout = jnp.dot(lw_ref[...], flat, preferred_element_type=_F32) + lb_ref[...]
    mu = jnp.sum(lout, axis=(0, 1), keepdims=True)[0] / float(Cout * S)
    xc = lout - mu
    var = jnp.sum(xc * xc, axis=(0, 1), keepdims=True)[0] / float(Cout * S)
    out_ref[0] = jnp.maximum(xc / jnp.sqrt(var + 1e-5) * gl_ref[...] + bel_ref[...], 0.0)


def _col(v):
    return v[:, None]


def _sa_weight_args(p, O, Cout):
    m = p['mlp'][0]
    w1, w2, w3 = p['weight']
    d1, d2, d3 = p['density']
    return [
        m['w'], _col(m['b']), _col(m['g']), _col(m['beta']),
        w1['w'], _col(w1['b']), _col(w1['g']), _col(w1['beta']),
        w2['w'], _col(w2['b']), _col(w2['g']), _col(w2['beta']),
        w3['w'], _col(w3['b']), _col(w3['g']), _col(w3['beta']),
        d1['w'], _col(d1['b']), _col(d1['g']), _col(d1['beta']),
        d2['w'], _col(d2['b']), _col(d2['g']), _col(d2['beta']),
        d3['w'], _col(d3['b']), _col(d3['g']), _col(d3['beta']),
        p['linear']['w'], _col(p['linear']['b']),
        _col(p['bn_linear']['g']), _col(p['bn_linear']['beta']),
    ]


def _w_specs(args, start):
    return [pl.BlockSpec(a.shape, lambda b: (0,) * a.ndim) for a in args[start:]]


def _sa_layer(p, xyzT, ptsT, npoint, nsample, bw, interpret=False, new_xyz=None):
    B, _, N = xyzT.shape
    C = ptsT.shape[1]
    S, K = npoint, nsample
    O = p['mlp'][0]['w'].shape[0]
    Cout = p['linear']['w'].shape[0]
    Ct = C + 4

    nxT = _fps_new_xyz(xyzT, S, interpret=interpret) if new_xyz is None else new_xyz
    xyzN = jnp.transpose(xyzT, (0, 2, 1))
    wargs = _sa_weight_args(p, O, Cout)

    in_specs = [
        pl.BlockSpec((1, N, 3), lambda b: (b, 0, 0)),
        pl.BlockSpec((1, 3, N), lambda b: (b, 0, 0)),
        pl.BlockSpec((1, C, N), lambda b: (b, 0, 0)),
        pl.BlockSpec((1, 3, S), lambda b: (b, 0, 0)),
    ] + [pl.BlockSpec(a.shape, functools.partial(lambda nd, b: (0,) * nd, a.ndim))
         for a in wargs]

    out = pl.pallas_call(
        functools.partial(_sa_body_kernel, N, S, K, C, O, Cout, bw),
        grid=(B,),
        in_specs=in_specs,
        out_specs=pl.BlockSpec((1, Cout, S), lambda b: (b, 0, 0)),
        out_shape=jax.ShapeDtypeStruct((B, Cout, S), _F32),
        scratch_shapes=[
            pltpu.VMEM((N, S), _F32),
            pltpu.VMEM((K, Ct, S), _F32),
            pltpu.VMEM((K, O, S), _F32),
            pltpu.VMEM((K, 8, S), _F32),
            pltpu.VMEM((K, 8, S), _F32),
            pltpu.VMEM((K, 16, S), _F32),
            pltpu.VMEM((K, 16, S), _F32),
            pltpu.VMEM((K, 8, S), _F32),
            pltpu.VMEM((K, 1, S), _F32),
            pltpu.VMEM((O, 16, S), _F32),
        ],
        compiler_params=pltpu.CompilerParams(dimension_semantics=("parallel",)),
        interpret=interpret,
    )(xyzN, xyzT, ptsT, nxT, *wargs)
    return nxT, out


def _sa_all_kernel(N, C, O, Cout, bw,
                   xyzN_ref, xyzT_ref, ptsT_ref,
                   wm_ref, bm_ref, gm_ref, bem_ref,
                   ww1_ref, bw1_ref, gw1_ref, bew1_ref,
                   ww2_ref, bw2_ref, gw2_ref, bew2_ref,
                   ww3_ref, bw3_ref, gw3_ref, bew3_ref,
                   wd1_ref, bd1_ref, gd1_ref, bed1_ref,
                   wd2_ref, bd2_ref, gd2_ref, bed2_ref,
                   wd3_ref, bd3_ref, gd3_ref, bed3_ref,
                   lw3_ref, lb_ref, gl_ref, bel_ref,
                   out_ref):
    xyzN = xyzN_ref[0]
    X = xyzT_ref[0]
    P = ptsT_ref[0]

    a_row = jnp.sum(X * X, axis=0, keepdims=True)
    a_col = jnp.sum(xyzN * xyzN, axis=1, keepdims=True)
    Gm = jnp.dot(xyzN, X, preferred_element_type=_F32)
    sqr_full = (a_col + a_row) - 2.0 * Gm
    gk = jnp.exp(-sqr_full / (2.0 * bw * bw)) / (2.5 * bw)
    invd = N / jnp.sum(gk, axis=0, keepdims=True)

    def gn_full(x, cnt, g, beta):
        mu = jnp.sum(x, axis=(0, 1), keepdims=True)[0] / cnt
        xc = x - mu
        var = jnp.sum(xc * xc, axis=(0, 1), keepdims=True)[0] / cnt
        return xc / jnp.sqrt(var + 1e-5) * g + beta

    npts = jnp.concatenate([X, P], axis=0)
    h = jnp.dot(wm_ref[...], npts, preferred_element_type=_F32) + bm_ref[...]
    h = jnp.maximum(gn_full(h, float(O * N), gm_ref[...], bem_ref[...]), 0.0)

    dsc = invd / jnp.max(invd, axis=1, keepdims=True)
    d1 = jnp.maximum(gn_full(wd1_ref[...] * dsc + bd1_ref[...], float(16 * N),
                             gd1_ref[...], bed1_ref[...]), 0.0)
    d2 = jnp.dot(wd2_ref[...], d1, preferred_element_type=_F32) + bd2_ref[...]
    d2 = jnp.maximum(gn_full(d2, float(8 * N), gd2_ref[...], bed2_ref[...]), 0.0)
    d3 = jnp.dot(wd3_ref[...], d2, preferred_element_type=_F32) + bd3_ref[...]
    d3 = _sigmoid(gn_full(d3, float(N), gd3_ref[...], bed3_ref[...]))

    x = h * d3

    w1 = jnp.dot(ww1_ref[...], X, preferred_element_type=_F32) + bw1_ref[...]
    w1 = jnp.maximum(gn_full(w1, float(8 * N), gw1_ref[...], bew1_ref[...]), 0.0)
    w2 = jnp.dot(ww2_ref[...], w1, preferred_element_type=_F32) + bw2_ref[...]
    w2 = jnp.maximum(gn_full(w2, float(8 * N), gw2_ref[...], bew2_ref[...]), 0.0)
    w3 = jnp.dot(ww3_ref[...], w2, preferred_element_type=_F32) + bw3_ref[...]
    wt = jnp.maximum(gn_full(w3, float(16 * N), gw3_ref[...], bew3_ref[...]), 0.0)

    y = jnp.zeros((Cout, N), _F32)
    for w in range(16):
        tw = jnp.dot(lw3_ref[w], x, preferred_element_type=_F32)
        y = y + tw * wt[w:w + 1, :]
    lout = jnp.sum(y, axis=1, keepdims=True) + lb_ref[...]

    mu = jnp.sum(lout, axis=(0, 1), keepdims=True)[0] / float(Cout)
    xc = lout - mu
    var = jnp.sum(xc * xc, axis=(0, 1), keepdims=True)[0] / float(Cout)
    out_ref[0] = jnp.maximum(xc / jnp.sqrt(var + 1e-5) * gl_ref[...] + bel_ref[...], 0.0)


def _sa_layer_all(p, xyzT, ptsT, bw, interpret=False):
    B, _, N = xyzT.shape
    C = ptsT.shape[1]
    O = p['mlp'][0]['w'].shape[0]
    Cout = p['linear']['w'].shape[0]

    wargs = _sa_weight_args(p, O, Cout)
    wargs[28] = jnp.transpose(wargs[28].reshape(Cout, O, 16), (2, 0, 1))
    xyzN = jnp.transpose(xyzT, (0, 2, 1))

    in_specs = [
        pl.BlockSpec((1, N, 3), lambda b: (b, 0, 0)),
        pl.BlockSpec((1, 3, N), lambda b: (b, 0, 0)),
        pl.BlockSpec((1, C, N), lambda b: (b, 0, 0)),
    ] + [pl.BlockSpec(a.shape, functools.partial(lambda nd, b: (0,) * nd, a.ndim))
         for a in wargs]

    out = pl.pallas_call(
        functools.partial(_sa_all_kernel, N, C, O, Cout, bw),
        grid=(B,),
        in_specs=in_specs,
        out_specs=pl.BlockSpec((1, Cout, 1), lambda b: (b, 0, 0)),
        out_shape=jax.ShapeDtypeStruct((B, Cout, 1), _F32),
        compiler_params=pltpu.CompilerParams(dimension_semantics=("parallel",)),
        interpret=interpret,
    )(xyzN, xyzT, ptsT, *wargs)
    return out[:, :, 0]


def _fc_head_kernel(h_ref, w1_ref, b1_ref, g1_ref, be1_ref,
                    w3_ref, b3_ref, g3_ref, be3_ref,
                    w4_ref, b4_ref, g4_ref, be4_ref,
                    w5_ref, b5_ref, out_ref):
    def gn_relu(x, g, b, eps=1e-5):
        mu = jnp.mean(x, axis=1, keepdims=True)
        var = jnp.mean((x - mu) ** 2, axis=1, keepdims=True)
        xh = (x - mu) / jnp.sqrt(var + eps)
        return jnp.maximum(xh * g[None, :] + b[None, :], 0.0)

    h = h_ref[...]
    h = gn_relu(jnp.dot(h, w1_ref[...].T, preferred_element_type=_F32) + b1_ref[...][None, :],
                g1_ref[...], be1_ref[...])
    h = gn_relu(jnp.dot(h, w3_ref[...].T, preferred_element_type=_F32) + b3_ref[...][None, :],
                g3_ref[...], be3_ref[...])
    h = gn_relu(jnp.dot(h, w4_ref[...].T, preferred_element_type=_F32) + b4_ref[...][None, :],
                g4_ref[...], be4_ref[...])
    out_ref[...] = jnp.dot(h, w5_ref[...].T, preferred_element_type=_F32) + b5_ref[...][None, :]


def _fc_head(params, h, interpret=False):
    B = h.shape[0]
    args = [h,
            params['fc1']['w'], params['fc1']['b'], params['bn1']['g'], params['bn1']['beta'],
            params['fc3']['w'], params['fc3']['b'], params['bn3']['g'], params['bn3']['beta'],
            params['fc4']['w'], params['fc4']['b'], params['bn4']['g'], params['bn4']['beta'],
            params['fc5']['w'], params['fc5']['b']]
    return pl.pallas_call(
        _fc_head_kernel,
        out_shape=jax.ShapeDtypeStruct((B, 6), jnp.float32),
        interpret=interpret,
    )(*args)


def _fps_pair(xyzT_a, xyzT_b, npoint, interpret=False):
    B = xyzT_a.shape[0]
    nx = _fps_new_xyz(jnp.concatenate([xyzT_a, xyzT_b], axis=0), npoint, interpret)
    return nx[:B], nx[B:]


def _forward(params, xyz, xyz_goal, interpret=False):
    nx1, ng1 = _fps_pair(xyz[:, :3, :], xyz_goal, 512, interpret)
    l1x, l1p = _sa_layer(params['sa1'], xyz[:, :3, :], xyz, 512, 32, 0.1, interpret,
                         new_xyz=nx1)
    g1x, g1p = _sa_layer(params['sa1_g'], xyz_goal, xyz_goal, 512, 32, 0.1, interpret,
                         new_xyz=ng1)
    nx2, ng2 = _fps_pair(l1x, g1x, 128, interpret)
    l2x, l2p = _sa_layer(params['sa2'], l1x, l1p, 128, 64, 0.2, interpret, new_xyz=nx2)
    g2x, g2p = _sa_layer(params['sa2_g'], g1x, g1p, 128, 64, 0.2, interpret, new_xyz=ng2)
    x = _sa_layer_all(params['sa3'], l2x, l2p, 0.4, interpret)
    g = _sa_layer_all(params['sa3_g'], g2x, g2p, 0.4, interpret)
    h = jnp.concatenate([x, g], axis=-1)
    return _fc_head(params, h, interpret)


def kernel(xyz, xyz_goal, params):
    return _forward(params, xyz, xyz_goal)
